# baseline (speedup 1.0000x reference)
import functools

import jax
import jax.numpy as jnp
from jax import lax
from jax.experimental import pallas as pl
from jax.experimental.pallas import tpu as pltpu
from jax.experimental.pallas import tpu_sc as plsc

N = 10000
NP = 10240
E = 320000
EP = 327680
TRASH = N
NC = 2
NS = 16
NW = NC * NS
EW = EP // NW
CH = 128
NCH = EW // CH
NG = 5
G = NCH // NG
RT = NP // NS
BLK = 1000
F32 = jnp.float32


def _mesh():
    return plsc.VectorSubcoreMesh(core_axis_name="c", subcore_axis_name="s")


def _deg_call(dst_r, zeros1):
    @functools.partial(
        pl.kernel,
        out_type=jax.ShapeDtypeStruct((NC, NP), F32),
        mesh=_mesh(),
        scratch_types=[
            pltpu.VMEM((NG, G, CH), jnp.int32),
            pltpu.VMEM((CH,), F32),
            pltpu.VMEM_SHARED((NP,), F32),
            pltpu.SemaphoreType.DMA,
        ],
    )
    def deg_k(dst_hbm, z_hbm, out_hbm, dst_v, ones_v, acc, sem):
        c = lax.axis_index("c")
        s = lax.axis_index("s")
        wid = s * NC + c
        pltpu.sync_copy(z_hbm.at[pl.ds(s * RT, RT)], acc.at[pl.ds(s * RT, RT)])
        pltpu.sync_copy(dst_hbm.at[wid], dst_v)
        for i in range(CH // 16):
            ones_v[pl.ds(i * 16, 16)] = jnp.ones((16,), F32)
        plsc.subcore_barrier()

        for gi in range(NG):
            def body(j, carry):
                pltpu.sync_copy(ones_v, acc.at[dst_v.at[gi, j]], add=True)
                return carry

            lax.fori_loop(0, G, body, 0)
        plsc.subcore_barrier()
        pltpu.sync_copy(acc.at[pl.ds(s * RT, RT)], out_hbm.at[c, pl.ds(s * RT, RT)])

    return deg_k(dst_r, zeros1)


def _scatter_call(y, src_r, dst_r, zerosf, f):
    @functools.partial(
        pl.kernel,
        out_type=jax.ShapeDtypeStruct((NC, NP, f), F32),
        mesh=_mesh(),
        scratch_types=[
            pltpu.VMEM((G, CH), jnp.int32),
            pltpu.VMEM((G, CH), jnp.int32),
            pltpu.VMEM((CH, f), F32),
            pltpu.VMEM((CH, f), F32),
            pltpu.VMEM_SHARED((NP, f), F32),
            pltpu.SemaphoreType.DMA,
            pltpu.SemaphoreType.DMA,
            pltpu.SemaphoreType.DMA,
            pltpu.SemaphoreType.DMA,
        ],
        compiler_params=pltpu.CompilerParams(use_tc_tiling_on_sc=(f == 128)),
    )
    def scat_k(y_hbm, src_hbm, dst_hbm, z_hbm, out_hbm,
               src_v, dst_v, r0, r1, acc, g0, g1, s0, s1):
        c = lax.axis_index("c")
        s = lax.axis_index("s")
        wid = s * NC + c
        pltpu.sync_copy(z_hbm.at[pl.ds(s * RT, RT)], acc.at[pl.ds(s * RT, RT)])
        plsc.subcore_barrier()

        def gather(j, r, sem):
            return pltpu.async_copy(y_hbm.at[src_v.at[j]], r, sem)

        def scat(j, r, sem):
            pltpu.async_copy(r, acc.at[dst_v.at[j]], sem, add=True)

        def drain(r, sem):
            pltpu.make_async_copy(r, acc.at[dst_v.at[0]], sem).wait()

        for gi in range(NG):
            pltpu.sync_copy(src_hbm.at[wid, gi], src_v)
            pltpu.sync_copy(dst_hbm.at[wid, gi], dst_v)

            h0 = gather(0, r0, g0)
            h1 = gather(1, r1, g1)
            h0.wait()
            scat(0, r0, s0)
            h1.wait()
            scat(1, r1, s1)

            def pair(i, carry):
                j0 = 2 * i
                drain(r0, s0)
                a0 = gather(j0, r0, g0)
                drain(r1, s1)
                a1 = gather(j0 + 1, r1, g1)
                a0.wait()
                scat(j0, r0, s0)
                a1.wait()
                scat(j0 + 1, r1, s1)
                return carry

            lax.fori_loop(1, G // 2, pair, 0)
            drain(r0, s0)
            drain(r1, s1)
        plsc.subcore_barrier()
        pltpu.sync_copy(acc.at[pl.ds(s * RT, RT)], out_hbm.at[c, pl.ds(s * RT, RT)])

    return scat_k(y, src_r, dst_r, zerosf)


def _tc1(deg_t, x, w1):
    def body(dref, xref, wref, yref, dinvref):
        d = dref[...]
        dinv = lax.rsqrt(1.0 + d[:, 0:1] + d[:, 1:2])
        yref[...] = jnp.dot(dinv * xref[...], wref[...], preferred_element_type=F32)
        dinvref[...] = dinv

    return pl.pallas_call(
        body,
        grid=(N // BLK,),
        in_specs=[
            pl.BlockSpec((BLK, 2), lambda i: (i, 0)),
            pl.BlockSpec((BLK, 128), lambda i: (i, 0)),
            pl.BlockSpec((128, 128), lambda i: (0, 0)),
        ],
        out_specs=[
            pl.BlockSpec((BLK, 128), lambda i: (i, 0)),
            pl.BlockSpec((BLK, 1), lambda i: (i, 0)),
        ],
        out_shape=[
            jax.ShapeDtypeStruct((N, 128), F32),
            jax.ShapeDtypeStruct((N, 1), F32),
        ],
    )(deg_t, x, w1)


def _tc2(s0, s1, y1, dinv, b1, w2):
    def body(s0r, s1r, y1r, dr, br, wr, outr):
        dv = dr[...]
        h = jnp.maximum(dv * (s0r[...] + s1r[...] + y1r[...]) + br[...], 0.0)
        outr[...] = jnp.dot(dv * h, wr[...], preferred_element_type=F32)

    return pl.pallas_call(
        body,
        grid=(N // BLK,),
        in_specs=[
            pl.BlockSpec((BLK, 128), lambda i: (i, 0)),
            pl.BlockSpec((BLK, 128), lambda i: (i, 0)),
            pl.BlockSpec((BLK, 128), lambda i: (i, 0)),
            pl.BlockSpec((BLK, 1), lambda i: (i, 0)),
            pl.BlockSpec((1, 128), lambda i: (0, 0)),
            pl.BlockSpec((128, 64), lambda i: (0, 0)),
        ],
        out_specs=pl.BlockSpec((BLK, 64), lambda i: (i, 0)),
        out_shape=jax.ShapeDtypeStruct((N, 64), F32),
    )(s0, s1, y1, dinv, b1, w2)


def _tc3(s0, s1, y2, dinv, b2):
    def body(s0r, s1r, y2r, dr, br, outr):
        outr[...] = dr[...] * (s0r[...] + s1r[...] + y2r[...]) + br[...]

    return pl.pallas_call(
        body,
        grid=(N // BLK,),
        in_specs=[
            pl.BlockSpec((BLK, 64), lambda i: (i, 0)),
            pl.BlockSpec((BLK, 64), lambda i: (i, 0)),
            pl.BlockSpec((BLK, 64), lambda i: (i, 0)),
            pl.BlockSpec((BLK, 1), lambda i: (i, 0)),
            pl.BlockSpec((1, 64), lambda i: (0, 0)),
        ],
        out_specs=pl.BlockSpec((BLK, 64), lambda i: (i, 0)),
        out_shape=jax.ShapeDtypeStruct((N, 64), F32),
    )(s0, s1, y2, dinv, b2)


def kernel(x, edge_index, W1, b1, W2, b2):
    pad = EP - E
    src_r = jnp.concatenate(
        [edge_index[0], jnp.zeros((pad,), jnp.int32)]).reshape(NW, NG, G, CH)
    trash = TRASH + jnp.arange(pad, dtype=jnp.int32) % (NP - N)
    dst_r = jnp.concatenate(
        [edge_index[1], trash]).reshape(NW, NG, G, CH)
    zeros1 = jnp.zeros((NP,), F32)
    zeros128 = jnp.zeros((NP, 128), F32)
    zeros64 = jnp.zeros((NP, 64), F32)

    degp = _deg_call(dst_r, zeros1)
    deg_t = jnp.transpose(degp)[:N]

    y1, dinv = _tc1(deg_t, x, W1)
    s1p = _scatter_call(y1, src_r, dst_r, zeros128, 128)
    y2 = _tc2(s1p[0, :N], s1p[1, :N], y1, dinv, b1.reshape(1, 128), W2)
    s2p = _scatter_call(y2, src_r, dst_r, zeros64, 64)
    return _tc3(s2p[0, :N], s2p[1, :N], y2, dinv, b2.reshape(1, 64))


# speedup vs baseline: 2.3793x; 2.3793x over previous
import functools

import jax
import jax.numpy as jnp
from jax import lax
from jax.experimental import pallas as pl
from jax.experimental.pallas import tpu as pltpu
from jax.experimental.pallas import tpu_sc as plsc

N = 10000
NP = 10240
E = 320000
EP = 320000
TRASH = N
NC = 2
NS = 16
NW = NC * NS
EW = EP // NW
CH = 80
NCH = EW // CH
NG = 5
G = NCH // NG
RT = NP // NS
BLK = 1000
F32 = jnp.float32


def _mesh():
    return plsc.VectorSubcoreMesh(core_axis_name="c", subcore_axis_name="s")


def _deg_call(dst_r, zeros1):
    @functools.partial(
        pl.kernel,
        out_type=jax.ShapeDtypeStruct((NC, NP), F32),
        mesh=_mesh(),
        scratch_types=[
            pltpu.VMEM((NG, G, CH), jnp.int32),
            pltpu.VMEM((CH,), F32),
            pltpu.VMEM_SHARED((NP,), F32),
            pltpu.SemaphoreType.DMA,
        ],
    )
    def deg_k(dst_hbm, z_hbm, out_hbm, dst_v, ones_v, acc, sem):
        c = lax.axis_index("c")
        s = lax.axis_index("s")
        wid = s * NC + c
        pltpu.sync_copy(z_hbm.at[pl.ds(s * RT, RT)], acc.at[pl.ds(s * RT, RT)])
        pltpu.sync_copy(dst_hbm.at[wid], dst_v)
        for i in range(CH // 16):
            ones_v[pl.ds(i * 16, 16)] = jnp.ones((16,), F32)
        plsc.subcore_barrier()

        for gi in range(NG):
            def body(j, carry):
                pltpu.sync_copy(ones_v, acc.at[dst_v.at[gi, j]], add=True)
                return carry

            lax.fori_loop(0, G, body, 0)
        plsc.subcore_barrier()
        pltpu.sync_copy(acc.at[pl.ds(s * RT, RT)], out_hbm.at[c, pl.ds(s * RT, RT)])

    return deg_k(dst_r, zeros1)


def _scatter_call(y, src_r, dst_r, zerosf, f):
    @functools.partial(
        pl.kernel,
        out_type=jax.ShapeDtypeStruct((NC, NP, f), F32),
        mesh=_mesh(),
        scratch_types=[
            pltpu.VMEM((G, CH), jnp.int32),
            pltpu.VMEM((G, CH), jnp.int32),
            pltpu.VMEM((CH, f), F32),
            pltpu.VMEM((CH, f), F32),
            pltpu.VMEM_SHARED((NP, f), F32),
            pltpu.SemaphoreType.DMA,
            pltpu.SemaphoreType.DMA,
            pltpu.SemaphoreType.DMA,
            pltpu.SemaphoreType.DMA,
        ],
        compiler_params=pltpu.CompilerParams(use_tc_tiling_on_sc=(f == 128)),
    )
    def scat_k(y_hbm, src_hbm, dst_hbm, z_hbm, out_hbm,
               src_v, dst_v, r0, r1, acc, g0, g1, s0, s1):
        c = lax.axis_index("c")
        s = lax.axis_index("s")
        wid = s * NC + c
        pltpu.sync_copy(z_hbm.at[pl.ds(s * RT, RT)], acc.at[pl.ds(s * RT, RT)])
        plsc.subcore_barrier()

        def gather(j, r, sem):
            return pltpu.async_copy(y_hbm.at[src_v.at[j]], r, sem)

        def scat(j, r, sem):
            pltpu.async_copy(r, acc.at[dst_v.at[j]], sem, add=True)

        def drain(r, sem):
            pltpu.make_async_copy(r, acc.at[dst_v.at[0]], sem).wait()

        for gi in range(NG):
            pltpu.sync_copy(src_hbm.at[wid, gi], src_v)
            pltpu.sync_copy(dst_hbm.at[wid, gi], dst_v)

            h0 = gather(0, r0, g0)
            h1 = gather(1, r1, g1)
            h0.wait()
            scat(0, r0, s0)
            h1.wait()
            scat(1, r1, s1)

            def pair(i, carry):
                j0 = 2 * i
                drain(r0, s0)
                a0 = gather(j0, r0, g0)
                drain(r1, s1)
                a1 = gather(j0 + 1, r1, g1)
                a0.wait()
                scat(j0, r0, s0)
                a1.wait()
                scat(j0 + 1, r1, s1)
                return carry

            lax.fori_loop(1, G // 2, pair, 0)
            if G % 2:
                drain(r0, s0)
                hT = gather(G - 1, r0, g0)
                hT.wait()
                scat(G - 1, r0, s0)
            drain(r0, s0)
            drain(r1, s1)
        plsc.subcore_barrier()
        pltpu.sync_copy(acc.at[pl.ds(s * RT, RT)], out_hbm.at[c, pl.ds(s * RT, RT)])

    return scat_k(y, src_r, dst_r, zerosf)


def _tc1(deg_t, x, w1):
    def body(dref, xref, wref, yref, dinvref):
        d = dref[...]
        dinv = lax.rsqrt(1.0 + d[:, 0:1] + d[:, 1:2])
        yref[...] = jnp.dot(dinv * xref[...], wref[...], preferred_element_type=F32)
        dinvref[...] = dinv

    return pl.pallas_call(
        body,
        grid=(N // BLK,),
        in_specs=[
            pl.BlockSpec((BLK, 2), lambda i: (i, 0)),
            pl.BlockSpec((BLK, 128), lambda i: (i, 0)),
            pl.BlockSpec((128, 128), lambda i: (0, 0)),
        ],
        out_specs=[
            pl.BlockSpec((BLK, 128), lambda i: (i, 0)),
            pl.BlockSpec((BLK, 1), lambda i: (i, 0)),
        ],
        out_shape=[
            jax.ShapeDtypeStruct((N, 128), F32),
            jax.ShapeDtypeStruct((N, 1), F32),
        ],
    )(deg_t, x, w1)


def _tc2(s0, s1, y1, dinv, b1, w2):
    def body(s0r, s1r, y1r, dr, br, wr, outr):
        dv = dr[...]
        h = jnp.maximum(dv * (s0r[...] + s1r[...] + y1r[...]) + br[...], 0.0)
        outr[...] = jnp.dot(dv * h, wr[...], preferred_element_type=F32)

    return pl.pallas_call(
        body,
        grid=(N // BLK,),
        in_specs=[
            pl.BlockSpec((BLK, 128), lambda i: (i, 0)),
            pl.BlockSpec((BLK, 128), lambda i: (i, 0)),
            pl.BlockSpec((BLK, 128), lambda i: (i, 0)),
            pl.BlockSpec((BLK, 1), lambda i: (i, 0)),
            pl.BlockSpec((1, 128), lambda i: (0, 0)),
            pl.BlockSpec((128, 64), lambda i: (0, 0)),
        ],
        out_specs=pl.BlockSpec((BLK, 64), lambda i: (i, 0)),
        out_shape=jax.ShapeDtypeStruct((N, 64), F32),
    )(s0, s1, y1, dinv, b1, w2)


def _tc3(s0, s1, y2, dinv, b2):
    def body(s0r, s1r, y2r, dr, br, outr):
        outr[...] = dr[...] * (s0r[...] + s1r[...] + y2r[...]) + br[...]

    return pl.pallas_call(
        body,
        grid=(N // BLK,),
        in_specs=[
            pl.BlockSpec((BLK, 64), lambda i: (i, 0)),
            pl.BlockSpec((BLK, 64), lambda i: (i, 0)),
            pl.BlockSpec((BLK, 64), lambda i: (i, 0)),
            pl.BlockSpec((BLK, 1), lambda i: (i, 0)),
            pl.BlockSpec((1, 64), lambda i: (0, 0)),
        ],
        out_specs=pl.BlockSpec((BLK, 64), lambda i: (i, 0)),
        out_shape=jax.ShapeDtypeStruct((N, 64), F32),
    )(s0, s1, y2, dinv, b2)


def kernel(x, edge_index, W1, b1, W2, b2):
    pad = EP - E
    src_r = jnp.concatenate(
        [edge_index[0], jnp.zeros((pad,), jnp.int32)]).reshape(NW, NG, G, CH)
    trash = TRASH + jnp.arange(pad, dtype=jnp.int32) % (NP - N)
    dst_r = jnp.concatenate(
        [edge_index[1], trash]).reshape(NW, NG, G, CH)
    zeros1 = jnp.zeros((NP,), F32)
    zeros128 = jnp.zeros((NP, 128), F32)
    zeros64 = jnp.zeros((NP, 64), F32)

    degp = _deg_call(dst_r, zeros1)
    deg_t = jnp.transpose(degp)[:N]

    y1, dinv = _tc1(deg_t, x, W1)
    s1p = _scatter_call(y1, src_r, dst_r, zeros128, 128)
    y2 = _tc2(s1p[0, :N], s1p[1, :N], y1, dinv, b1.reshape(1, 128), W2)
    s2p = _scatter_call(y2, src_r, dst_r, zeros64, 64)
    return _tc3(s2p[0, :N], s2p[1, :N], y2, dinv, b2.reshape(1, 64))


# speedup vs baseline: 2.4720x; 1.0389x over previous
import functools

import jax
import jax.numpy as jnp
from jax import lax
from jax.experimental import pallas as pl
from jax.experimental.pallas import tpu as pltpu
from jax.experimental.pallas import tpu_sc as plsc

N = 10000
NP = 10240
E = 320000
EP = 320000
TRASH = N
NC = 2
NS = 16
NW = NC * NS
EW = EP // NW
CH = 80
NCH = EW // CH
NG = 5
G = NCH // NG
RT = NP // NS
BLK = 1000
F32 = jnp.float32


def _mesh():
    return plsc.VectorSubcoreMesh(core_axis_name="c", subcore_axis_name="s")


def _deg_call(dst_r, zeros1):
    @functools.partial(
        pl.kernel,
        out_type=jax.ShapeDtypeStruct((NC, NP), F32),
        mesh=_mesh(),
        scratch_types=[
            pltpu.VMEM((NG, G, CH), jnp.int32),
            pltpu.VMEM((CH,), F32),
            pltpu.VMEM_SHARED((NP,), F32),
            pltpu.SemaphoreType.DMA,
        ],
    )
    def deg_k(dst_hbm, z_hbm, out_hbm, dst_v, ones_v, acc, sem):
        c = lax.axis_index("c")
        s = lax.axis_index("s")
        wid = s * NC + c
        pltpu.sync_copy(z_hbm.at[pl.ds(s * RT, RT)], acc.at[pl.ds(s * RT, RT)])
        pltpu.sync_copy(dst_hbm.at[wid], dst_v)
        for i in range(CH // 16):
            ones_v[pl.ds(i * 16, 16)] = jnp.ones((16,), F32)
        plsc.subcore_barrier()

        for gi in range(NG):
            def body(j, carry):
                pltpu.sync_copy(ones_v, acc.at[dst_v.at[gi, j]], add=True)
                return carry

            lax.fori_loop(0, G, body, 0)
        plsc.subcore_barrier()
        pltpu.sync_copy(acc.at[pl.ds(s * RT, RT)], out_hbm.at[c, pl.ds(s * RT, RT)])

    return deg_k(dst_r, zeros1)


def _scatter_call(y, src_r, dst_r, zerosf, f):
    @functools.partial(
        pl.kernel,
        out_type=jax.ShapeDtypeStruct((NC, NP, f), F32),
        mesh=_mesh(),
        scratch_types=[
            pltpu.VMEM((G, CH), jnp.int32),
            pltpu.VMEM((G, CH), jnp.int32),
            pltpu.VMEM((CH, f), F32),
            pltpu.VMEM((CH, f), F32),
            pltpu.VMEM_SHARED((NP, f), F32),
            pltpu.SemaphoreType.DMA,
            pltpu.SemaphoreType.DMA,
            pltpu.SemaphoreType.DMA,
            pltpu.SemaphoreType.DMA,
        ],
        compiler_params=pltpu.CompilerParams(use_tc_tiling_on_sc=(f == 128)),
    )
    def scat_k(y_hbm, src_hbm, dst_hbm, z_hbm, out_hbm,
               src_v, dst_v, r0, r1, acc, g0, g1, s0, s1):
        c = lax.axis_index("c")
        s = lax.axis_index("s")
        wid = s * NC + c
        pltpu.sync_copy(z_hbm.at[pl.ds(s * RT, RT)], acc.at[pl.ds(s * RT, RT)])
        plsc.subcore_barrier()

        def gather(j, r, sem):
            return pltpu.async_copy(y_hbm.at[src_v.at[j]], r, sem)

        def scat(j, r, sem):
            pltpu.async_copy(r, acc.at[dst_v.at[j]], sem, add=True)

        def drain(r, sem):
            pltpu.make_async_copy(r, acc.at[dst_v.at[0]], sem).wait()

        for gi in range(NG):
            pltpu.sync_copy(src_hbm.at[wid, gi], src_v)
            pltpu.sync_copy(dst_hbm.at[wid, gi], dst_v)

            h0 = gather(0, r0, g0)
            h1 = gather(1, r1, g1)
            h0.wait()
            scat(0, r0, s0)
            h1.wait()
            scat(1, r1, s1)

            def pair(i, carry):
                j0 = 2 * i
                drain(r0, s0)
                a0 = gather(j0, r0, g0)
                drain(r1, s1)
                a1 = gather(j0 + 1, r1, g1)
                a0.wait()
                scat(j0, r0, s0)
                a1.wait()
                scat(j0 + 1, r1, s1)
                return carry

            lax.fori_loop(1, G // 2, pair, 0)
            if G % 2:
                drain(r0, s0)
                hT = gather(G - 1, r0, g0)
                hT.wait()
                scat(G - 1, r0, s0)
            drain(r0, s0)
            drain(r1, s1)
        plsc.subcore_barrier()
        pltpu.sync_copy(acc.at[pl.ds(s * RT, RT)], out_hbm.at[c, pl.ds(s * RT, RT)])

    return scat_k(y, src_r, dst_r, zerosf)


def _tc0(x, w1):
    def body(xref, wref, yref):
        yref[...] = jnp.dot(xref[...], wref[...], preferred_element_type=F32)

    return pl.pallas_call(
        body,
        grid=(N // BLK,),
        in_specs=[
            pl.BlockSpec((BLK, 128), lambda i: (i, 0)),
            pl.BlockSpec((128, 128), lambda i: (0, 0)),
        ],
        out_specs=pl.BlockSpec((BLK, 128), lambda i: (i, 0)),
        out_shape=jax.ShapeDtypeStruct((N, 128), F32),
    )(x, w1)


def _tc1(deg_t, xw):
    def body(dref, xwref, yref, dinvref):
        d = dref[...]
        dinv = lax.rsqrt(1.0 + d[:, 0:1] + d[:, 1:2])
        yref[...] = dinv * xwref[...]
        dinvref[...] = dinv

    return pl.pallas_call(
        body,
        grid=(N // BLK,),
        in_specs=[
            pl.BlockSpec((BLK, 2), lambda i: (i, 0)),
            pl.BlockSpec((BLK, 128), lambda i: (i, 0)),
        ],
        out_specs=[
            pl.BlockSpec((BLK, 128), lambda i: (i, 0)),
            pl.BlockSpec((BLK, 1), lambda i: (i, 0)),
        ],
        out_shape=[
            jax.ShapeDtypeStruct((N, 128), F32),
            jax.ShapeDtypeStruct((N, 1), F32),
        ],
    )(deg_t, xw)


def _tc2(s1p, y1, dinv, b1, w2):
    def body(s0r, s1r, y1r, dr, br, wr, outr):
        dv = dr[...]
        h = jnp.maximum(dv * (s0r[0] + s1r[0] + y1r[...]) + br[...], 0.0)
        outr[...] = jnp.dot(dv * h, wr[...], preferred_element_type=F32)

    return pl.pallas_call(
        body,
        grid=(N // BLK,),
        in_specs=[
            pl.BlockSpec((1, BLK, 128), lambda i: (0, i, 0)),
            pl.BlockSpec((1, BLK, 128), lambda i: (1, i, 0)),
            pl.BlockSpec((BLK, 128), lambda i: (i, 0)),
            pl.BlockSpec((BLK, 1), lambda i: (i, 0)),
            pl.BlockSpec((1, 128), lambda i: (0, 0)),
            pl.BlockSpec((128, 64), lambda i: (0, 0)),
        ],
        out_specs=pl.BlockSpec((BLK, 64), lambda i: (i, 0)),
        out_shape=jax.ShapeDtypeStruct((N, 64), F32),
    )(s1p, s1p, y1, dinv, b1, w2)


def _tc3(s2p, y2, dinv, b2):
    def body(s0r, s1r, y2r, dr, br, outr):
        outr[...] = dr[...] * (s0r[0] + s1r[0] + y2r[...]) + br[...]

    return pl.pallas_call(
        body,
        grid=(N // BLK,),
        in_specs=[
            pl.BlockSpec((1, BLK, 64), lambda i: (0, i, 0)),
            pl.BlockSpec((1, BLK, 64), lambda i: (1, i, 0)),
            pl.BlockSpec((BLK, 64), lambda i: (i, 0)),
            pl.BlockSpec((BLK, 1), lambda i: (i, 0)),
            pl.BlockSpec((1, 64), lambda i: (0, 0)),
        ],
        out_specs=pl.BlockSpec((BLK, 64), lambda i: (i, 0)),
        out_shape=jax.ShapeDtypeStruct((N, 64), F32),
    )(s2p, s2p, y2, dinv, b2)


def kernel(x, edge_index, W1, b1, W2, b2):
    src_r = edge_index[0].reshape(NW, NG, G, CH)
    dst_r = edge_index[1].reshape(NW, NG, G, CH)
    zeros1 = jnp.zeros((NP,), F32)
    zeros128 = jnp.zeros((NP, 128), F32)
    zeros64 = jnp.zeros((NP, 64), F32)

    degp = _deg_call(dst_r, zeros1)
    deg_t = jnp.transpose(degp)[:N]

    xw = _tc0(x, W1)
    y1, dinv = _tc1(deg_t, xw)
    s1p = _scatter_call(y1, src_r, dst_r, zeros128, 128)
    y2 = _tc2(s1p, y1, dinv, b1.reshape(1, 128), W2)
    s2p = _scatter_call(y2, src_r, dst_r, zeros64, 64)
    return _tc3(s2p, y2, dinv, b2.reshape(1, 64))


# speedup vs baseline: 2.9096x; 1.1771x over previous
import functools

import jax
import jax.numpy as jnp
from jax import lax
from jax.experimental import pallas as pl
from jax.experimental.pallas import tpu as pltpu
from jax.experimental.pallas import tpu_sc as plsc

N = 10000
NP = 10240
E = 320000
EP = 320000
TRASH = N
NC = 2
NS = 16
NW = NC * NS
EW = EP // NW
CH = 80
NCH = EW // CH
NG = 5
G = NCH // NG
RT = NP // NS
BLK = 1000
F32 = jnp.float32


def _mesh():
    return plsc.VectorSubcoreMesh(core_axis_name="c", subcore_axis_name="s")


def _deg_call(dst_r, zeros1):
    @functools.partial(
        pl.kernel,
        out_type=jax.ShapeDtypeStruct((NC, NP), F32),
        mesh=_mesh(),
        scratch_types=[
            pltpu.VMEM((NG, G, CH), jnp.int32),
            pltpu.VMEM((CH,), F32),
            pltpu.VMEM_SHARED((NP,), F32),
            pltpu.SemaphoreType.DMA,
        ],
    )
    def deg_k(dst_hbm, z_hbm, out_hbm, dst_v, ones_v, acc, sem):
        c = lax.axis_index("c")
        s = lax.axis_index("s")
        wid = s * NC + c
        pltpu.sync_copy(z_hbm.at[pl.ds(s * RT, RT)], acc.at[pl.ds(s * RT, RT)])
        pltpu.sync_copy(dst_hbm.at[wid], dst_v)
        for i in range(CH // 16):
            ones_v[pl.ds(i * 16, 16)] = jnp.ones((16,), F32)
        plsc.subcore_barrier()

        for gi in range(NG):
            def body(j, carry):
                pltpu.sync_copy(ones_v, acc.at[dst_v.at[gi, j]], add=True)
                return carry

            lax.fori_loop(0, G, body, 0)
        plsc.subcore_barrier()
        pltpu.sync_copy(acc.at[pl.ds(s * RT, RT)], out_hbm.at[c, pl.ds(s * RT, RT)])

    return deg_k(dst_r, zeros1)


def _scatter_call(y, src_r, dst_r, zerosf, f):
    @functools.partial(
        pl.kernel,
        out_type=jax.ShapeDtypeStruct((NC, NP, f), F32),
        mesh=_mesh(),
        scratch_types=[
            pltpu.VMEM((2, G, CH), jnp.int32),
            pltpu.VMEM((2, G, CH), jnp.int32),
            pltpu.VMEM((CH, f), F32),
            pltpu.VMEM((CH, f), F32),
            pltpu.VMEM((CH, f), F32),
            pltpu.VMEM_SHARED((NP, f), F32),
            pltpu.SemaphoreType.DMA,
            pltpu.SemaphoreType.DMA,
            pltpu.SemaphoreType.DMA,
            pltpu.SemaphoreType.DMA,
            pltpu.SemaphoreType.DMA,
            pltpu.SemaphoreType.DMA,
            pltpu.SemaphoreType.DMA,
            pltpu.SemaphoreType.DMA,
        ],
        compiler_params=pltpu.CompilerParams(use_tc_tiling_on_sc=(f == 128)),
    )
    def scat_k(y_hbm, src_hbm, dst_hbm, z_hbm, out_hbm,
               src_v, dst_v, r0, r1, r2, acc,
               g0, g1, g2, s0, s1, s2, i0, i1):
        c = lax.axis_index("c")
        s = lax.axis_index("s")
        wid = s * NC + c
        pltpu.sync_copy(z_hbm.at[pl.ds(s * RT, RT)], acc.at[pl.ds(s * RT, RT)])
        plsc.subcore_barrier()

        pltpu.sync_copy(src_hbm.at[wid, 0], src_v.at[0])
        pltpu.sync_copy(dst_hbm.at[wid, 0], dst_v.at[0])
        for gi in range(NG):
            p = gi % 2
            if gi + 1 < NG:
                pf0 = pltpu.async_copy(src_hbm.at[wid, gi + 1], src_v.at[1 - p], i0)
                pf1 = pltpu.async_copy(dst_hbm.at[wid, gi + 1], dst_v.at[1 - p], i1)

            def gather(j, r, sem):
                return pltpu.async_copy(y_hbm.at[src_v.at[p, j]], r, sem)

            def scat(j, r, sem):
                pltpu.async_copy(r, acc.at[dst_v.at[p, j]], sem, add=True)

            def drain(r, sem):
                pltpu.make_async_copy(r, acc.at[dst_v.at[p, 0]], sem).wait()

            h0 = gather(0, r0, g0)
            h1 = gather(1, r1, g1)
            h2 = gather(2, r2, g2)
            h0.wait()
            scat(0, r0, s0)
            h1.wait()
            scat(1, r1, s1)
            h2.wait()
            scat(2, r2, s2)

            def trip(i, carry):
                j = 3 * i
                drain(r0, s0)
                a0 = gather(j, r0, g0)
                drain(r1, s1)
                a1 = gather(j + 1, r1, g1)
                drain(r2, s2)
                a2 = gather(j + 2, r2, g2)
                a0.wait()
                scat(j, r0, s0)
                a1.wait()
                scat(j + 1, r1, s1)
                a2.wait()
                scat(j + 2, r2, s2)
                return carry

            lax.fori_loop(1, G // 3, trip, 0)
            for j in range(3 * (G // 3), G):
                drain(r0, s0)
                hT = gather(j, r0, g0)
                hT.wait()
                scat(j, r0, s0)
            drain(r0, s0)
            drain(r1, s1)
            drain(r2, s2)
            if gi + 1 < NG:
                pf0.wait()
                pf1.wait()
        plsc.subcore_barrier()
        pltpu.sync_copy(acc.at[pl.ds(s * RT, RT)], out_hbm.at[c, pl.ds(s * RT, RT)])

    return scat_k(y, src_r, dst_r, zerosf)


def _tc0(x, w1):
    def body(xref, wref, yref):
        yref[...] = jnp.dot(xref[...], wref[...], preferred_element_type=F32)

    return pl.pallas_call(
        body,
        grid=(N // BLK,),
        in_specs=[
            pl.BlockSpec((BLK, 128), lambda i: (i, 0)),
            pl.BlockSpec((128, 128), lambda i: (0, 0)),
        ],
        out_specs=pl.BlockSpec((BLK, 128), lambda i: (i, 0)),
        out_shape=jax.ShapeDtypeStruct((N, 128), F32),
    )(x, w1)


def _tc1(deg_t, xw):
    def body(dref, xwref, yref, dinvref):
        d = dref[...]
        dinv = lax.rsqrt(1.0 + d[:, 0:1] + d[:, 1:2])
        yref[...] = dinv * xwref[...]
        dinvref[...] = dinv

    return pl.pallas_call(
        body,
        grid=(N // BLK,),
        in_specs=[
            pl.BlockSpec((BLK, 2), lambda i: (i, 0)),
            pl.BlockSpec((BLK, 128), lambda i: (i, 0)),
        ],
        out_specs=[
            pl.BlockSpec((BLK, 128), lambda i: (i, 0)),
            pl.BlockSpec((BLK, 1), lambda i: (i, 0)),
        ],
        out_shape=[
            jax.ShapeDtypeStruct((N, 128), F32),
            jax.ShapeDtypeStruct((N, 1), F32),
        ],
    )(deg_t, xw)


def _tc2(s1p, y1, dinv, b1, w2):
    def body(s0r, s1r, y1r, dr, br, wr, outr):
        dv = dr[...]
        h = jnp.maximum(dv * (s0r[0] + s1r[0] + y1r[...]) + br[...], 0.0)
        outr[...] = jnp.dot(dv * h, wr[...], preferred_element_type=F32)

    return pl.pallas_call(
        body,
        grid=(N // BLK,),
        in_specs=[
            pl.BlockSpec((1, BLK, 128), lambda i: (0, i, 0)),
            pl.BlockSpec((1, BLK, 128), lambda i: (1, i, 0)),
            pl.BlockSpec((BLK, 128), lambda i: (i, 0)),
            pl.BlockSpec((BLK, 1), lambda i: (i, 0)),
            pl.BlockSpec((1, 128), lambda i: (0, 0)),
            pl.BlockSpec((128, 64), lambda i: (0, 0)),
        ],
        out_specs=pl.BlockSpec((BLK, 64), lambda i: (i, 0)),
        out_shape=jax.ShapeDtypeStruct((N, 64), F32),
    )(s1p, s1p, y1, dinv, b1, w2)


def _tc3(s2p, y2, dinv, b2):
    def body(s0r, s1r, y2r, dr, br, outr):
        outr[...] = dr[...] * (s0r[0] + s1r[0] + y2r[...]) + br[...]

    return pl.pallas_call(
        body,
        grid=(N // BLK,),
        in_specs=[
            pl.BlockSpec((1, BLK, 64), lambda i: (0, i, 0)),
            pl.BlockSpec((1, BLK, 64), lambda i: (1, i, 0)),
            pl.BlockSpec((BLK, 64), lambda i: (i, 0)),
            pl.BlockSpec((BLK, 1), lambda i: (i, 0)),
            pl.BlockSpec((1, 64), lambda i: (0, 0)),
        ],
        out_specs=pl.BlockSpec((BLK, 64), lambda i: (i, 0)),
        out_shape=jax.ShapeDtypeStruct((N, 64), F32),
    )(s2p, s2p, y2, dinv, b2)


def kernel(x, edge_index, W1, b1, W2, b2):
    src_r = edge_index[0].reshape(NW, NG, G, CH)
    dst_r = edge_index[1].reshape(NW, NG, G, CH)
    zeros1 = jnp.zeros((NP,), F32)
    zeros128 = jnp.zeros((NP, 128), F32)
    zeros64 = jnp.zeros((NP, 64), F32)

    degp = _deg_call(dst_r, zeros1)
    deg_t = jnp.transpose(degp)[:N]

    xw = _tc0(x, W1)
    y1, dinv = _tc1(deg_t, xw)
    s1p = _scatter_call(y1, src_r, dst_r, zeros128, 128)
    y2 = _tc2(s1p, y1, dinv, b1.reshape(1, 128), W2)
    s2p = _scatter_call(y2, src_r, dst_r, zeros64, 64)
    return _tc3(s2p, y2, dinv, b2.reshape(1, 64))


# speedup vs baseline: 3.0373x; 1.0439x over previous
import functools

import jax
import jax.numpy as jnp
from jax import lax
from jax.experimental import pallas as pl
from jax.experimental.pallas import tpu as pltpu
from jax.experimental.pallas import tpu_sc as plsc

N = 10000
NP = 10240
E = 320000
EP = 320000
TRASH = N
NC = 2
NS = 16
NW = NC * NS
EW = EP // NW
CH = 80
NCH = EW // CH
NG = 5
G = NCH // NG
RT = NP // NS
BLK = 1000
F32 = jnp.float32


def _mesh():
    return plsc.VectorSubcoreMesh(core_axis_name="c", subcore_axis_name="s")


def _deg_call(dst_r, zeros1):
    @functools.partial(
        pl.kernel,
        out_type=jax.ShapeDtypeStruct((NC, NP), F32),
        mesh=_mesh(),
        scratch_types=[
            pltpu.VMEM((NG, G, CH), jnp.int32),
            pltpu.VMEM((CH,), F32),
            pltpu.VMEM_SHARED((NP,), F32),
            pltpu.SemaphoreType.DMA,
        ],
    )
    def deg_k(dst_hbm, z_hbm, out_hbm, dst_v, ones_v, acc, sem):
        c = lax.axis_index("c")
        s = lax.axis_index("s")
        wid = s * NC + c
        pltpu.sync_copy(z_hbm.at[pl.ds(s * RT, RT)], acc.at[pl.ds(s * RT, RT)])
        pltpu.sync_copy(dst_hbm.at[wid], dst_v)
        for i in range(CH // 16):
            ones_v[pl.ds(i * 16, 16)] = jnp.ones((16,), F32)
        plsc.subcore_barrier()

        for gi in range(NG):
            def body(j, carry):
                pltpu.sync_copy(ones_v, acc.at[dst_v.at[gi, j]], add=True)
                return carry

            lax.fori_loop(0, G, body, 0)
        plsc.subcore_barrier()
        pltpu.sync_copy(acc.at[pl.ds(s * RT, RT)], out_hbm.at[c, pl.ds(s * RT, RT)])

    return deg_k(dst_r, zeros1)


def _scatter_call(y, src_r, dst_r, zerosf, f, nbuf):
    @functools.partial(
        pl.kernel,
        out_type=jax.ShapeDtypeStruct((NC, NP, f), F32),
        mesh=_mesh(),
        scratch_types=(
            [pltpu.VMEM((2, G, CH), jnp.int32)] * 2
            + [pltpu.VMEM((CH, f), F32)] * nbuf
            + [pltpu.VMEM_SHARED((NP, f), F32)]
            + [pltpu.SemaphoreType.DMA] * (2 * nbuf + 2)
        ),
        compiler_params=pltpu.CompilerParams(use_tc_tiling_on_sc=(f == 128)),
    )
    def scat_k(y_hbm, src_hbm, dst_hbm, z_hbm, out_hbm, src_v, dst_v, *rest):
        bufs = list(rest[:nbuf])
        acc = rest[nbuf]
        gs = list(rest[nbuf + 1:2 * nbuf + 1])
        ss = list(rest[2 * nbuf + 1:3 * nbuf + 1])
        i0, i1 = rest[3 * nbuf + 1], rest[3 * nbuf + 2]
        c = lax.axis_index("c")
        s = lax.axis_index("s")
        wid = s * NC + c
        pltpu.sync_copy(z_hbm.at[pl.ds(s * RT, RT)], acc.at[pl.ds(s * RT, RT)])
        plsc.subcore_barrier()

        pltpu.sync_copy(src_hbm.at[wid, 0], src_v.at[0])
        pltpu.sync_copy(dst_hbm.at[wid, 0], dst_v.at[0])
        for gi in range(NG):
            p = gi % 2
            if gi + 1 < NG:
                pf0 = pltpu.async_copy(src_hbm.at[wid, gi + 1], src_v.at[1 - p], i0)
                pf1 = pltpu.async_copy(dst_hbm.at[wid, gi + 1], dst_v.at[1 - p], i1)

            def gather(j, r, sem):
                return pltpu.async_copy(y_hbm.at[src_v.at[p, j]], r, sem)

            def scat(j, r, sem):
                pltpu.async_copy(r, acc.at[dst_v.at[p, j]], sem, add=True)

            def drain(r, sem):
                pltpu.make_async_copy(r, acc.at[dst_v.at[p, 0]], sem).wait()

            hs = [gather(b, bufs[b], gs[b]) for b in range(nbuf)]
            for b in range(nbuf):
                hs[b].wait()
                scat(b, bufs[b], ss[b])

            def rot(i, carry):
                j = nbuf * i
                aa = []
                for b in range(nbuf):
                    drain(bufs[b], ss[b])
                    aa.append(gather(j + b, bufs[b], gs[b]))
                for b in range(nbuf):
                    aa[b].wait()
                    scat(j + b, bufs[b], ss[b])
                return carry

            lax.fori_loop(1, G // nbuf, rot, 0)
            for j in range(nbuf * (G // nbuf), G):
                drain(bufs[0], ss[0])
                hT = gather(j, bufs[0], gs[0])
                hT.wait()
                scat(j, bufs[0], ss[0])
            for b in range(nbuf):
                drain(bufs[b], ss[b])
            if gi + 1 < NG:
                pf0.wait()
                pf1.wait()
        plsc.subcore_barrier()
        pltpu.sync_copy(acc.at[pl.ds(s * RT, RT)], out_hbm.at[c, pl.ds(s * RT, RT)])

    return scat_k(y, src_r, dst_r, zerosf)


def _tc0(x, w1):
    def body(xref, wref, yref):
        yref[...] = jnp.dot(xref[...], wref[...], preferred_element_type=F32)

    return pl.pallas_call(
        body,
        grid=(N // BLK,),
        in_specs=[
            pl.BlockSpec((BLK, 128), lambda i: (i, 0)),
            pl.BlockSpec((128, 128), lambda i: (0, 0)),
        ],
        out_specs=pl.BlockSpec((BLK, 128), lambda i: (i, 0)),
        out_shape=jax.ShapeDtypeStruct((N, 128), F32),
    )(x, w1)


def _tc1(deg_t, xw):
    def body(dref, xwref, yref, dinvref):
        d = dref[...]
        dinv = lax.rsqrt(1.0 + d[:, 0:1] + d[:, 1:2])
        yref[...] = dinv * xwref[...]
        dinvref[...] = dinv

    return pl.pallas_call(
        body,
        grid=(N // BLK,),
        in_specs=[
            pl.BlockSpec((BLK, 2), lambda i: (i, 0)),
            pl.BlockSpec((BLK, 128), lambda i: (i, 0)),
        ],
        out_specs=[
            pl.BlockSpec((BLK, 128), lambda i: (i, 0)),
            pl.BlockSpec((BLK, 1), lambda i: (i, 0)),
        ],
        out_shape=[
            jax.ShapeDtypeStruct((N, 128), F32),
            jax.ShapeDtypeStruct((N, 1), F32),
        ],
    )(deg_t, xw)


def _tc2(s1p, y1, dinv, b1, w2):
    def body(s0r, s1r, y1r, dr, br, wr, outr):
        dv = dr[...]
        h = jnp.maximum(dv * (s0r[0] + s1r[0] + y1r[...]) + br[...], 0.0)
        outr[...] = jnp.dot(dv * h, wr[...], preferred_element_type=F32)

    return pl.pallas_call(
        body,
        grid=(N // BLK,),
        in_specs=[
            pl.BlockSpec((1, BLK, 128), lambda i: (0, i, 0)),
            pl.BlockSpec((1, BLK, 128), lambda i: (1, i, 0)),
            pl.BlockSpec((BLK, 128), lambda i: (i, 0)),
            pl.BlockSpec((BLK, 1), lambda i: (i, 0)),
            pl.BlockSpec((1, 128), lambda i: (0, 0)),
            pl.BlockSpec((128, 64), lambda i: (0, 0)),
        ],
        out_specs=pl.BlockSpec((BLK, 64), lambda i: (i, 0)),
        out_shape=jax.ShapeDtypeStruct((N, 64), F32),
    )(s1p, s1p, y1, dinv, b1, w2)


def _tc3(s2p, y2, dinv, b2):
    def body(s0r, s1r, y2r, dr, br, outr):
        outr[...] = dr[...] * (s0r[0] + s1r[0] + y2r[...]) + br[...]

    return pl.pallas_call(
        body,
        grid=(N // BLK,),
        in_specs=[
            pl.BlockSpec((1, BLK, 64), lambda i: (0, i, 0)),
            pl.BlockSpec((1, BLK, 64), lambda i: (1, i, 0)),
            pl.BlockSpec((BLK, 64), lambda i: (i, 0)),
            pl.BlockSpec((BLK, 1), lambda i: (i, 0)),
            pl.BlockSpec((1, 64), lambda i: (0, 0)),
        ],
        out_specs=pl.BlockSpec((BLK, 64), lambda i: (i, 0)),
        out_shape=jax.ShapeDtypeStruct((N, 64), F32),
    )(s2p, s2p, y2, dinv, b2)


def kernel(x, edge_index, W1, b1, W2, b2):
    src_r = edge_index[0].reshape(NW, NG, G, CH)
    dst_r = edge_index[1].reshape(NW, NG, G, CH)
    zeros1 = jnp.zeros((NP,), F32)
    zeros128 = jnp.zeros((NP, 128), F32)
    zeros64 = jnp.zeros((NP, 64), F32)

    degp = _deg_call(dst_r, zeros1)
    deg_t = jnp.transpose(degp)[:N]

    xw = _tc0(x, W1)
    y1, dinv = _tc1(deg_t, xw)
    s1p = _scatter_call(y1, src_r, dst_r, zeros128, 128, 3)
    y2 = _tc2(s1p, y1, dinv, b1.reshape(1, 128), W2)
    s2p = _scatter_call(y2, src_r, dst_r, zeros64, 64, 5)
    return _tc3(s2p, y2, dinv, b2.reshape(1, 64))
